# speedup vs baseline: 18.8449x; 18.8449x over previous
import functools

import jax
import jax.numpy as jnp
from jax import lax
from jax.experimental import pallas as pl

C = 256
F32 = jnp.float32
I32 = jnp.int32


def _full(shape):
    return pl.BlockSpec(shape, lambda b: tuple(0 for _ in shape))


def _prep_body(x0a, x2, W_all, b_all, Wp2a, Wp2b, W2a3, cvec2, Wp1a, M1,
               feat2_o, T1_o):
    f2 = jnp.dot(x0a[0], W_all[...], preferred_element_type=F32) + b_all[...]
    cls2 = jnp.max(f2, axis=0, keepdims=True)
    cls_t = jnp.dot(cls2, Wp2a[...], preferred_element_type=F32)
    feat2 = (cls_t
             + jnp.dot(x2[0], W2a3[...], preferred_element_type=F32)
             + jnp.dot(f2, Wp2b[...], preferred_element_type=F32)
             + cvec2[...])
    feat2_o[0] = feat2
    T1_o[0] = (jnp.dot(feat2, Wp1a[...], preferred_element_type=F32)
               - jnp.dot(x2[0], M1[...], preferred_element_type=F32))


def _argmin_body(pwd_ref, idx_o, *, n_rows, n_cols):
    g = n_rows // 128
    vals = pwd_ref[0].reshape(g, 128, n_cols)
    m = jnp.min(vals, axis=2, keepdims=True)
    iota = lax.broadcasted_iota(I32, (g, 128, n_cols), 2)
    cand = jnp.where(vals == m, iota, n_cols)
    idx_o[0] = jnp.min(cand, axis=2)


def _mid_body(T1, idx3, x1, x0b, M1, A1, c1, Wp0a, M0, feat1_o, T0_o):
    idx = idx3[0]
    iota = lax.broadcasted_iota(I32, (4, 128, 128), 2)
    oh = (iota == idx[:, :, None]).astype(F32).reshape(512, 128)
    G1 = jnp.dot(oh, T1[0], preferred_element_type=F32)
    feat1 = (G1
             + jnp.dot(x1[0], M1[...], preferred_element_type=F32)
             + jnp.dot(x0b[0], A1[...], preferred_element_type=F32)
             + c1[...])
    feat1_o[0] = feat1
    T0_o[0] = (jnp.dot(feat1, Wp0a[...], preferred_element_type=F32)
               - jnp.dot(x1[0], M0[...], preferred_element_type=F32))


def _final_body(T0, idx3, x0, A0, c0, feat0_o):
    idx = idx3[0]
    iota = lax.broadcasted_iota(I32, (16, 128, 512), 2)
    oh = (iota == idx[:, :, None]).astype(F32).reshape(2048, 512)
    G0 = jnp.dot(oh, T0[0], preferred_element_type=F32)
    feat0_o[0] = (G0
                  + jnp.dot(x0[0], A0[...], preferred_element_type=F32)
                  + c0[...])


def kernel(xyz0, xyz1, xyz2, pwd, W_all, b_all, W2, b2, W1, b1, W0, b0,
           Wp2, bp2, Wp1, bp1, Wp0, bp0):
    B, N0, _ = xyz0.shape
    N1 = xyz1.shape[1]
    N2 = xyz2.shape[1]

    Wp2a, Wp2b = Wp2[:C], Wp2[C:]
    Wp1a, Wp1b = Wp1[:C], Wp1[C:]
    Wp0a, Wp0b = Wp0[:C], Wp0[C:]
    W2a3 = W2 @ Wp2a
    cvec2 = (b2 @ Wp2a + bp2)[None, :]
    M1 = W1 @ Wp1a
    A1 = W_all @ Wp1b
    c1 = (b1 @ Wp1a + b_all @ Wp1b + bp1)[None, :]
    M0 = W0 @ Wp0a
    A0 = M0 + W_all @ Wp0b
    c0 = (b0 @ Wp0a + b_all @ Wp0b + bp0)[None, :]
    b_all2 = b_all[None, :]

    feat2, T1 = pl.pallas_call(
        _prep_body,
        grid=(B,),
        in_specs=[
            pl.BlockSpec((1, N2, 3), lambda b: (b, 0, 0)),
            pl.BlockSpec((1, N2, 3), lambda b: (b, 0, 0)),
            _full((3, C)), _full((1, C)), _full((C, C)), _full((C, C)),
            _full((3, C)), _full((1, C)), _full((C, C)), _full((3, C)),
        ],
        out_specs=[
            pl.BlockSpec((1, N2, C), lambda b: (b, 0, 0)),
            pl.BlockSpec((1, N2, C), lambda b: (b, 0, 0)),
        ],
        out_shape=[
            jax.ShapeDtypeStruct((B, N2, C), F32),
            jax.ShapeDtypeStruct((B, N2, C), F32),
        ],
    )(xyz0[:, :N2], xyz2, W_all, b_all2, Wp2a, Wp2b, W2a3, cvec2, Wp1a, M1)

    idx12 = pl.pallas_call(
        functools.partial(_argmin_body, n_rows=N1, n_cols=N2),
        grid=(B,),
        in_specs=[pl.BlockSpec((1, N1, N2), lambda b: (b, 0, 0))],
        out_specs=pl.BlockSpec((1, N1 // 128, 128), lambda b: (b, 0, 0)),
        out_shape=jax.ShapeDtypeStruct((B, N1 // 128, 128), I32),
    )(pwd[:, :N1, :N2])

    idx01 = pl.pallas_call(
        functools.partial(_argmin_body, n_rows=N0, n_cols=N1),
        grid=(B,),
        in_specs=[pl.BlockSpec((1, N0, N1), lambda b: (b, 0, 0))],
        out_specs=pl.BlockSpec((1, N0 // 128, 128), lambda b: (b, 0, 0)),
        out_shape=jax.ShapeDtypeStruct((B, N0 // 128, 128), I32),
    )(pwd[:, :N0, :N1])

    feat1, T0 = pl.pallas_call(
        _mid_body,
        grid=(B,),
        in_specs=[
            pl.BlockSpec((1, N2, C), lambda b: (b, 0, 0)),
            pl.BlockSpec((1, N1 // 128, 128), lambda b: (b, 0, 0)),
            pl.BlockSpec((1, N1, 3), lambda b: (b, 0, 0)),
            pl.BlockSpec((1, N1, 3), lambda b: (b, 0, 0)),
            _full((3, C)), _full((3, C)), _full((1, C)), _full((C, C)),
            _full((3, C)),
        ],
        out_specs=[
            pl.BlockSpec((1, N1, C), lambda b: (b, 0, 0)),
            pl.BlockSpec((1, N1, C), lambda b: (b, 0, 0)),
        ],
        out_shape=[
            jax.ShapeDtypeStruct((B, N1, C), F32),
            jax.ShapeDtypeStruct((B, N1, C), F32),
        ],
    )(T1, idx12, xyz1, xyz0[:, :N1], M1, A1, c1, Wp0a, M0)

    feat0 = pl.pallas_call(
        _final_body,
        grid=(B,),
        in_specs=[
            pl.BlockSpec((1, N1, C), lambda b: (b, 0, 0)),
            pl.BlockSpec((1, N0 // 128, 128), lambda b: (b, 0, 0)),
            pl.BlockSpec((1, N0, 3), lambda b: (b, 0, 0)),
            _full((3, C)), _full((1, C)),
        ],
        out_specs=pl.BlockSpec((1, N0, C), lambda b: (b, 0, 0)),
        out_shape=jax.ShapeDtypeStruct((B, N0, C), F32),
    )(T0, idx01, xyz0, A0, c0)

    return (feat2, feat1, feat0)


# speedup vs baseline: 24.7669x; 1.3142x over previous
import functools

import jax
import jax.numpy as jnp
from jax import lax
from jax.experimental import pallas as pl

C = 256
F32 = jnp.float32
I32 = jnp.int32


def _full(shape):
    return pl.BlockSpec(shape, lambda b: tuple(0 for _ in shape))


def _prep_body(x0a, x2, W_all, b_all, Wp2a, Wp2b, W2a3, cvec2, Wp1a, M1,
               feat2_o, T1_o):
    f2 = jnp.dot(x0a[0], W_all[...], preferred_element_type=F32) + b_all[...]
    cls2 = jnp.max(f2, axis=0, keepdims=True)
    cls_t = jnp.dot(cls2, Wp2a[...], preferred_element_type=F32)
    feat2 = (cls_t
             + jnp.dot(x2[0], W2a3[...], preferred_element_type=F32)
             + jnp.dot(f2, Wp2b[...], preferred_element_type=F32)
             + cvec2[...])
    feat2_o[0] = feat2
    T1_o[0] = (jnp.dot(feat2, Wp1a[...], preferred_element_type=F32)
               - jnp.dot(x2[0], M1[...], preferred_element_type=F32))


def _argmin_body(pwd_ref, idx_o, *, n_rows, n_cols):
    g = n_rows // 128
    vals = pwd_ref[0].reshape(g, 128, n_cols)
    m = jnp.min(vals, axis=2, keepdims=True)
    iota = lax.broadcasted_iota(I32, (g, 128, n_cols), 2)
    cand = jnp.where(vals == m, iota, n_cols)
    idx_o[0] = jnp.min(cand, axis=2)


def _mid_body(T1, idx3, x1, x0b, M1, A1, c1, Wp0a, M0, feat1_o, T0_o):
    idx = idx3[0]
    iota = lax.broadcasted_iota(I32, (4, 128, 128), 2)
    oh = (iota == idx[:, :, None]).astype(F32).reshape(512, 128)
    G1 = jnp.dot(oh, T1[0], preferred_element_type=F32)
    feat1 = (G1
             + jnp.dot(x1[0], M1[...], preferred_element_type=F32)
             + jnp.dot(x0b[0], A1[...], preferred_element_type=F32)
             + c1[...])
    feat1_o[0] = feat1
    T0_o[0] = (jnp.dot(feat1, Wp0a[...], preferred_element_type=F32)
               - jnp.dot(x1[0], M0[...], preferred_element_type=F32))


def _final_body(T0, idx3, x0, A0, c0, feat0_o):
    idx = idx3[0]
    iota = lax.broadcasted_iota(I32, (16, 128, 512), 2)
    oh = (iota == idx[:, :, None]).astype(F32).reshape(2048, 512)
    G0 = jnp.dot(oh, T0[0], preferred_element_type=F32)
    feat0_o[0] = (G0
                  + jnp.dot(x0[0], A0[...], preferred_element_type=F32)
                  + c0[...])


def kernel(xyz0, xyz1, xyz2, pwd, W_all, b_all, W2, b2, W1, b1, W0, b0,
           Wp2, bp2, Wp1, bp1, Wp0, bp0):
    B, N0, _ = xyz0.shape
    N1 = xyz1.shape[1]
    N2 = xyz2.shape[1]

    Wp2a, Wp2b = Wp2[:C], Wp2[C:]
    Wp1a, Wp1b = Wp1[:C], Wp1[C:]
    Wp0a, Wp0b = Wp0[:C], Wp0[C:]
    W2a3 = W2 @ Wp2a
    cvec2 = (b2 @ Wp2a + bp2)[None, :]
    M1 = W1 @ Wp1a
    A1 = W_all @ Wp1b
    c1 = (b1 @ Wp1a + b_all @ Wp1b + bp1)[None, :]
    M0 = W0 @ Wp0a
    A0 = M0 + W_all @ Wp0b
    c0 = (b0 @ Wp0a + b_all @ Wp0b + bp0)[None, :]
    b_all2 = b_all[None, :]

    feat2, T1 = pl.pallas_call(
        _prep_body,
        grid=(B,),
        in_specs=[
            pl.BlockSpec((1, N2, 3), lambda b: (b, 0, 0)),
            pl.BlockSpec((1, N2, 3), lambda b: (b, 0, 0)),
            _full((3, C)), _full((1, C)), _full((C, C)), _full((C, C)),
            _full((3, C)), _full((1, C)), _full((C, C)), _full((3, C)),
        ],
        out_specs=[
            pl.BlockSpec((1, N2, C), lambda b: (b, 0, 0)),
            pl.BlockSpec((1, N2, C), lambda b: (b, 0, 0)),
        ],
        out_shape=[
            jax.ShapeDtypeStruct((B, N2, C), F32),
            jax.ShapeDtypeStruct((B, N2, C), F32),
        ],
    )(xyz0, xyz2, W_all, b_all2, Wp2a, Wp2b, W2a3, cvec2, Wp1a, M1)

    idx12 = pl.pallas_call(
        functools.partial(_argmin_body, n_rows=N1, n_cols=N2),
        grid=(B,),
        in_specs=[pl.BlockSpec((1, N1, N2), lambda b: (b, 0, 0))],
        out_specs=pl.BlockSpec((1, N1 // 128, 128), lambda b: (b, 0, 0)),
        out_shape=jax.ShapeDtypeStruct((B, N1 // 128, 128), I32),
    )(pwd)

    idx01 = pl.pallas_call(
        functools.partial(_argmin_body, n_rows=N0, n_cols=N1),
        grid=(B,),
        in_specs=[pl.BlockSpec((1, N0, N1), lambda b: (b, 0, 0))],
        out_specs=pl.BlockSpec((1, N0 // 128, 128), lambda b: (b, 0, 0)),
        out_shape=jax.ShapeDtypeStruct((B, N0 // 128, 128), I32),
    )(pwd)

    feat1, T0 = pl.pallas_call(
        _mid_body,
        grid=(B,),
        in_specs=[
            pl.BlockSpec((1, N2, C), lambda b: (b, 0, 0)),
            pl.BlockSpec((1, N1 // 128, 128), lambda b: (b, 0, 0)),
            pl.BlockSpec((1, N1, 3), lambda b: (b, 0, 0)),
            pl.BlockSpec((1, N1, 3), lambda b: (b, 0, 0)),
            _full((3, C)), _full((3, C)), _full((1, C)), _full((C, C)),
            _full((3, C)),
        ],
        out_specs=[
            pl.BlockSpec((1, N1, C), lambda b: (b, 0, 0)),
            pl.BlockSpec((1, N1, C), lambda b: (b, 0, 0)),
        ],
        out_shape=[
            jax.ShapeDtypeStruct((B, N1, C), F32),
            jax.ShapeDtypeStruct((B, N1, C), F32),
        ],
    )(T1, idx12, xyz1, xyz0, M1, A1, c1, Wp0a, M0)

    feat0 = pl.pallas_call(
        _final_body,
        grid=(B,),
        in_specs=[
            pl.BlockSpec((1, N1, C), lambda b: (b, 0, 0)),
            pl.BlockSpec((1, N0 // 128, 128), lambda b: (b, 0, 0)),
            pl.BlockSpec((1, N0, 3), lambda b: (b, 0, 0)),
            _full((3, C)), _full((1, C)),
        ],
        out_specs=pl.BlockSpec((1, N0, C), lambda b: (b, 0, 0)),
        out_shape=jax.ShapeDtypeStruct((B, N0, C), F32),
    )(T0, idx01, xyz0, A0, c0)

    return (feat2, feat1, feat0)
